# baseline (speedup 1.0000x reference)
import functools

import jax
import jax.numpy as jnp
from jax import lax
from jax.experimental import pallas as pl
from jax.experimental.pallas import tpu as pltpu
from jax.experimental.pallas import tpu_sc as plsc

N_CORES = 2
N_SUBCORES = 16
NW = N_CORES * N_SUBCORES
LANES = 16

B = 16384
D = 32
BPW = B // NW
CHUNK = 128
NCHUNK = BPW // CHUNK
GPC = CHUNK // LANES

UF_ROWS = 250000
BIAS_ROWS = 7813


def _mf_body(users_hbm, items_hbm, uf_hbm, if_hbm, ub_hbm, ib_hbm, gb_hbm,
             out_hbm, uidx_v, iidx_v, urow_v, irow_v, ubrow_v, ibrow_v,
             ufat_v, ifat_v, ubias_v, ibias_v, gb_v, out_v, fsem, bsem):
    wid = lax.axis_index("s") * N_CORES + lax.axis_index("c")
    base = wid * BPW
    pltpu.sync_copy(users_hbm.at[pl.ds(base, BPW)], uidx_v)
    pltpu.sync_copy(items_hbm.at[pl.ds(base, BPW)], iidx_v)

    gb_v[...] = jnp.zeros((LANES,), jnp.float32)
    pltpu.sync_copy(gb_hbm, gb_v.at[pl.ds(0, 1)])

    lanes = lax.iota(jnp.int32, LANES)

    for c in range(NCHUNK):
        csplat = jnp.full((LANES,), c, jnp.int32)
        for k in range(GPC):
            src = lanes + (c * CHUNK + k * LANES)
            dst = lanes + k * LANES
            u = plsc.load_gather(uidx_v, [src])
            i = plsc.load_gather(iidx_v, [src])
            plsc.store_scatter(urow_v, [csplat, dst], u >> 2)
            plsc.store_scatter(irow_v, [csplat, dst], i >> 2)
            plsc.store_scatter(ubrow_v, [csplat, dst], u >> 7)
            plsc.store_scatter(ibrow_v, [csplat, dst], i >> 7)

    gbs = jnp.sum(gb_v[...])

    def fire_factors(c, slot):
        return (
            pltpu.async_copy(uf_hbm.at[urow_v.at[c]], ufat_v.at[slot], fsem),
            pltpu.async_copy(if_hbm.at[irow_v.at[c]], ifat_v.at[slot], fsem),
        )

    def fire_biases(c):
        return (
            pltpu.async_copy(ub_hbm.at[ubrow_v.at[c]], ubias_v, bsem),
            pltpu.async_copy(ib_hbm.at[ibrow_v.at[c]], ibias_v, bsem),
        )

    inflight_f = fire_factors(0, 0)
    inflight_b = fire_biases(0)

    for c in range(NCHUNK):
        for cp in inflight_f:
            cp.wait()
        if c + 1 < NCHUNK:
            next_f = fire_factors(c + 1, (c + 1) % 2)
        else:
            next_f = ()
        for cp in inflight_b:
            cp.wait()
        slot = c % 2
        srow = jnp.full((LANES,), slot, jnp.int32)
        for g in range(GPC):
            e_in_chunk = lanes + g * LANES
            src = e_in_chunk + c * CHUNK
            u = plsc.load_gather(uidx_v, [src])
            i = plsc.load_gather(iidx_v, [src])
            ucol = (u & 3) << 5
            icol = (i & 3) << 5
            acc = (plsc.load_gather(ubias_v, [e_in_chunk, u & 127])
                   + plsc.load_gather(ibias_v, [e_in_chunk, i & 127]) + gbs)
            for j in range(D):
                uv = plsc.load_gather(ufat_v, [srow, e_in_chunk, ucol + j])
                iv = plsc.load_gather(ifat_v, [srow, e_in_chunk, icol + j])
                acc = acc + uv * iv
            plsc.store_scatter(out_v, [src], acc)
        if c + 1 < NCHUNK:
            inflight_b = fire_biases(c + 1)
        inflight_f = next_f

    pltpu.sync_copy(out_v, out_hbm.at[pl.ds(base, BPW)])


@functools.partial(
    pl.kernel,
    out_type=jax.ShapeDtypeStruct((B,), jnp.float32),
    mesh=plsc.VectorSubcoreMesh(core_axis_name="c", subcore_axis_name="s"),
    compiler_params=pltpu.CompilerParams(needs_layout_passes=False),
    scratch_types=[
        pltpu.VMEM((BPW,), jnp.int32),
        pltpu.VMEM((BPW,), jnp.int32),
        pltpu.VMEM((NCHUNK, CHUNK), jnp.int32),
        pltpu.VMEM((NCHUNK, CHUNK), jnp.int32),
        pltpu.VMEM((NCHUNK, CHUNK), jnp.int32),
        pltpu.VMEM((NCHUNK, CHUNK), jnp.int32),
        pltpu.VMEM((2, CHUNK, 128), jnp.float32),
        pltpu.VMEM((2, CHUNK, 128), jnp.float32),
        pltpu.VMEM((CHUNK, 128), jnp.float32),
        pltpu.VMEM((CHUNK, 128), jnp.float32),
        pltpu.VMEM((LANES,), jnp.float32),
        pltpu.VMEM((BPW,), jnp.float32),
        pltpu.SemaphoreType.DMA,
        pltpu.SemaphoreType.DMA,
    ],
)
def _mf_kernel(*refs):
    _mf_body(*refs)


def kernel(data, user_factors, item_factors, user_bias, item_bias, global_bias):
    users = data[:, 0]
    items = data[:, 1]
    uf4 = user_factors.reshape(UF_ROWS, 128)
    if4 = item_factors.reshape(UF_ROWS, 128)
    ubp = jnp.pad(user_bias[:, 0], (0, BIAS_ROWS * 128 - user_bias.shape[0]))
    ibp = jnp.pad(item_bias[:, 0], (0, BIAS_ROWS * 128 - item_bias.shape[0]))
    ub2 = ubp.reshape(BIAS_ROWS, 128)
    ib2 = ibp.reshape(BIAS_ROWS, 128)
    return _mf_kernel(users, items, uf4, if4, ub2, ib2, global_bias)
